# baseline (speedup 1.0000x reference)
import functools

import jax
import jax.numpy as jnp
from jax import lax
from jax.experimental import pallas as pl
from jax.experimental.pallas import tpu as pltpu
from jax.experimental.pallas import tpu_sc as plsc

B = 4
T = 8192
D = 4096
K = T // 2
ALPHA = 0.1

NC = 2
NS = 16
L = 16
CHUNK = T // NS
OUTC = K // NS
BPC = B // NC

TBLK = 256


def _matvec_body(x_ref, w_ref, l_ref, g_ref, th_ref, rem_ref, gacc):
    t = pl.program_id(0)
    xb = x_ref[...].astype(jnp.bfloat16).astype(jnp.float32)
    w = w_ref[...].astype(jnp.bfloat16).astype(jnp.float32)
    logit = jnp.sum(xb * w[None], axis=2)
    gate = jax.nn.sigmoid(logit) * ALPHA
    l_ref[...] = logit
    g_ref[...] = gate
    gacc[:, pl.ds(t * TBLK, TBLK)] = gate

    @pl.when(t == T // TBLK - 1)
    def _():
        keys = lax.bitcast_convert_type(gacc[...], jnp.int32)

        def step(_, carry):
            lo, hi = carry
            mid = lo + (hi - lo + 1) // 2
            cnt = jnp.sum((keys >= mid).astype(jnp.int32), axis=1,
                          keepdims=True)
            take = cnt >= K
            return jnp.where(take, mid, lo), jnp.where(take, hi, mid - 1)

        lo0 = jnp.zeros((B, 1), jnp.int32)
        hi0 = jnp.full((B, 1), 0x7F800000, jnp.int32)
        lo, _ = lax.fori_loop(0, 32, step, (lo0, hi0))
        cnt_gt = jnp.sum((keys > lo).astype(jnp.int32), axis=1, keepdims=True)
        th_ref[...] = jnp.broadcast_to(
            lax.bitcast_convert_type(lo, jnp.float32), (B, L))
        rem_ref[...] = jnp.broadcast_to(K - cnt_gt, (B, L))


def _count_splat(m):
    return jnp.broadcast_to(jnp.sum(m.astype(jnp.int32)), (L,))


def _sc_select_body(gates_hbm, th_hbm, rem_hbm, sel_hbm, gsel_hbm,
                    gbuf, thv, remv, loc_i, loc_g, cnts_v, tmp16,
                    pfx_v, all_i, all_g, outb_i, outb_g, sems,
                    cnt_pub, idx_pub, gate_pub):
    c = lax.axis_index("c")
    s = lax.axis_index("s")
    w = c * NS + s
    iota = lax.iota(jnp.int32, L)
    zero16 = jnp.zeros((L,), jnp.int32)
    rows = c * NS + iota

    cps = [
        pltpu.async_copy(
            gates_hbm.at[pl.ds((c * BPC + 0) * T + s * CHUNK, CHUNK)],
            gbuf.at[pl.ds(0, CHUNK)], sems.at[0]),
        pltpu.async_copy(
            gates_hbm.at[pl.ds((c * BPC + 1) * T + s * CHUNK, CHUNK)],
            gbuf.at[pl.ds(CHUNK, CHUNK)], sems.at[1]),
        pltpu.async_copy(th_hbm.at[pl.ds(c * BPC * L, BPC * L)], thv,
                         sems.at[2]),
        pltpu.async_copy(rem_hbm.at[pl.ds(c * BPC * L, BPC * L)], remv,
                         sems.at[3]),
    ]
    cps[2].wait()
    cps[3].wait()

    def count_scan(q):
        th = thv[pl.ds(q * L, L)]

        def cnt_body(i, cc):
            cg, ce = cc
            g = gbuf[pl.ds(q * CHUNK + i * L, L)]
            cg = cg + _count_splat(g > th)
            ce = ce + _count_splat(g == th)
            return cg, ce

        return lax.fori_loop(0, CHUNK // L, cnt_body, (zero16, zero16))

    cps[0].wait()
    cg0, ce0 = count_scan(0)
    cps[1].wait()
    cg1, ce1 = count_scan(1)
    tmp16[...] = jnp.where(
        iota == 0, cg0, jnp.where(iota == 1, ce0,
                                  jnp.where(iota == 2, cg1,
                                            jnp.where(iota == 3, ce1,
                                                      zero16))))
    pltpu.sync_copy(tmp16, cnt_pub.at[w])
    plsc.subcore_barrier()

    pltpu.sync_copy(cnt_pub, cnts_v)

    def compact(q):
        th = thv[pl.ds(q * L, L)]
        rem = remv[pl.ds(q * L, L)]
        gtc = plsc.load_gather(cnts_v, [rows, zero16 + 2 * q])
        eqc = plsc.load_gather(cnts_v, [rows, zero16 + 2 * q + 1])
        eq_excl = plsc.cumsum(eqc) - eqc
        scv = gtc + jnp.clip(rem - eq_excl, 0, eqc)
        pfx_v[pl.ds(q * NS, NS)] = plsc.cumsum(scv) - scv
        tmp16[...] = eq_excl
        eq_base = plsc.load_gather(tmp16, [jnp.broadcast_to(s, (L,))])

        def sel_body(i, cc):
            pos_run, eq_run = cc
            g = gbuf[pl.ds(q * CHUNK + i * L, L)]
            mgt = g > th
            meq = g == th
            eqr = eq_run + plsc.cumsum(jnp.where(meq, 1, 0)) - 1
            m = mgt | (meq & (eq_base + eqr < rem))
            r = pos_run + plsc.cumsum(jnp.where(m, 1, 0)) - 1
            tok = (s * CHUNK + i * L) + iota
            plsc.store_scatter(loc_i, [q * CHUNK + r], tok, mask=m)
            plsc.store_scatter(loc_g, [q * CHUNK + r], g, mask=m)
            pos_run = pos_run + _count_splat(m)
            eq_run = eq_run + _count_splat(meq)
            return pos_run, eq_run

        lax.fori_loop(0, CHUNK // L, sel_body, (zero16, zero16))

    compact(0)
    compact(1)
    cpo = [pltpu.async_copy(loc_i, idx_pub.at[w], sems.at[0]),
           pltpu.async_copy(loc_g, gate_pub.at[w], sems.at[1])]
    cpo[0].wait()
    cpo[1].wait()
    plsc.subcore_barrier()

    start = s * OUTC
    pvec0 = pfx_v[pl.ds(0, NS)]
    pvec1 = pfx_v[pl.ds(NS, NS)]
    for u in range(NS):
        lo0 = pvec0[u]
        hi0 = K if u == NS - 1 else pvec0[u + 1]
        lo1 = pvec1[u]
        hi1 = K if u == NS - 1 else pvec1[u + 1]
        need0 = jnp.logical_and(lo0 < start + OUTC, hi0 > start)
        need1 = jnp.logical_and(lo1 < start + OUTC, hi1 > start)

        @pl.when(jnp.logical_or(need0, need1))
        def _():
            pltpu.sync_copy(idx_pub.at[c * NS + u],
                            all_i.at[pl.ds(u * BPC * CHUNK, BPC * CHUNK)])
            pltpu.sync_copy(gate_pub.at[c * NS + u],
                            all_g.at[pl.ds(u * BPC * CHUNK, BPC * CHUNK)])

    def merge(q):
        for v in range(OUTC // L):
            j = (start + v * L) + iota
            src = zero16
            for step in (8, 4, 2, 1):
                cand = src + step
                val = plsc.load_gather(pfx_v, [zero16 + q * NS + cand])
                src = jnp.where(val <= j, cand, src)
            off = jnp.clip(
                j - plsc.load_gather(pfx_v, [zero16 + q * NS + src]),
                0, CHUNK - 1)
            flat = (src * BPC + q) * CHUNK + off
            outb_i[pl.ds(q * OUTC + v * L, L)] = plsc.load_gather(
                all_i, [flat])
            outb_g[pl.ds(q * OUTC + v * L, L)] = plsc.load_gather(
                all_g, [flat])
        out_base = (c * BPC + q) * K + s * OUTC
        return [pltpu.async_copy(outb_i.at[pl.ds(q * OUTC, OUTC)],
                                 sel_hbm.at[pl.ds(out_base, OUTC)],
                                 sems.at[2 * q]),
                pltpu.async_copy(outb_g.at[pl.ds(q * OUTC, OUTC)],
                                 gsel_hbm.at[pl.ds(out_base, OUTC)],
                                 sems.at[2 * q + 1])]

    outs = merge(0) + merge(1)
    for cp in outs:
        cp.wait()


@functools.lru_cache(maxsize=1)
def _build_sc_select():
    return functools.partial(
        pl.kernel,
        out_type=[jax.ShapeDtypeStruct((B * K,), jnp.int32),
                  jax.ShapeDtypeStruct((B * K,), jnp.float32)],
        mesh=plsc.VectorSubcoreMesh(core_axis_name="c", subcore_axis_name="s",
                                    num_cores=NC, num_subcores=NS),
        compiler_params=pltpu.CompilerParams(needs_layout_passes=False),
        scratch_types=[
            pltpu.VMEM((BPC * CHUNK,), jnp.float32),
            pltpu.VMEM((BPC * L,), jnp.float32),
            pltpu.VMEM((BPC * L,), jnp.int32),
            pltpu.VMEM((BPC * CHUNK,), jnp.int32),
            pltpu.VMEM((BPC * CHUNK,), jnp.float32),
            pltpu.VMEM((NC * NS, L), jnp.int32),
            pltpu.VMEM((L,), jnp.int32),
            pltpu.VMEM((BPC * NS,), jnp.int32),
            pltpu.VMEM((NS * BPC * CHUNK,), jnp.int32),
            pltpu.VMEM((NS * BPC * CHUNK,), jnp.float32),
            pltpu.VMEM((BPC * OUTC,), jnp.int32),
            pltpu.VMEM((BPC * OUTC,), jnp.float32),
            pltpu.SemaphoreType.DMA((6,)),
            pltpu.VMEM_SHARED((NC * NS, L), jnp.int32),
            pltpu.VMEM_SHARED((NC * NS, BPC * CHUNK), jnp.int32),
            pltpu.VMEM_SHARED((NC * NS, BPC * CHUNK), jnp.float32),
        ],
    )(_sc_select_body)


def kernel(x, W):
    logits, gates, th, rem = pl.pallas_call(
        _matvec_body,
        grid=(T // TBLK,),
        in_specs=[
            pl.BlockSpec((B, TBLK, D), lambda t: (0, t, 0)),
            pl.BlockSpec((1, D), lambda t: (0, 0)),
        ],
        out_specs=[
            pl.BlockSpec((B, TBLK), lambda t: (0, t)),
            pl.BlockSpec((B, TBLK), lambda t: (0, t)),
            pl.BlockSpec((B, L), lambda t: (0, 0)),
            pl.BlockSpec((B, L), lambda t: (0, 0)),
        ],
        out_shape=[
            jax.ShapeDtypeStruct((B, T), jnp.float32),
            jax.ShapeDtypeStruct((B, T), jnp.float32),
            jax.ShapeDtypeStruct((B, L), jnp.float32),
            jax.ShapeDtypeStruct((B, L), jnp.int32),
        ],
        scratch_shapes=[pltpu.VMEM((B, T), jnp.float32)],
    )(x, W)

    sel, gsel = _build_sc_select()(gates.reshape(B * T),
                                   th.reshape(B * L), rem.reshape(B * L))
    selected_tokens = sel.reshape(B, K, 1).astype(jnp.int64)
    gate_weights = gsel.reshape(B, K, 1)
    raw_logits = logits.reshape(B, T, 1)
    return selected_tokens, gate_weights, raw_logits
